# baseline (speedup 1.0000x reference)
import functools

import jax
import jax.numpy as jnp
from jax import lax
from jax.experimental import pallas as pl
from jax.experimental.pallas import tpu as pltpu
from jax.experimental.pallas import tpu_sc as plsc

N = 10000
NP = 10240
HID = 128
E = 320000
CH = 128
PT0 = 80
PT1 = 80
PH = 16
TCH = 16 * (PT0 + PT1)
EPAD = TCH * CH
NTRASH = 240
RPS = NP // 16
BLK = 1024
F32 = jnp.float32
HIGH = lax.Precision.HIGHEST

@functools.cache
def _mesh():
    return plsc.VectorSubcoreMesh(core_axis_name="c", subcore_axis_name="s",
                                  num_cores=2, num_subcores=16)


def _edge_partition():
    cid = lax.axis_index("c")
    sid = lax.axis_index("s")
    ptc = jnp.where(cid == 0, PT0, PT1)
    nph = ptc // PH
    cbase = cid * (16 * PT0) + sid * ptc
    return cid, sid, cbase, nph


def _deg_body(dst_hbm, out_hbm, didx, ones, zb, dsh):
    cid, sid, cbase, nph = _edge_partition()

    def init_ones(i, c):
        ones[pl.ds(i * 16, 16)] = jnp.ones((16,), F32)
        return c

    lax.fori_loop(0, CH // 16, init_ones, 0)

    def init_zb(i, c):
        zb[pl.ds(i * 16, 16)] = jnp.zeros((16,), F32)
        return c

    lax.fori_loop(0, RPS // 16, init_zb, 0)
    pltpu.sync_copy(zb, dsh.at[pl.ds(sid * RPS, RPS)])
    plsc.subcore_barrier()

    def phase(p, c):
        pltpu.sync_copy(dst_hbm.at[pl.ds(cbase + p * PH, PH)], didx)

        def chunk(g, c2):
            pltpu.sync_copy(ones, dsh.at[didx.at[g]], add=True)
            return c2

        lax.fori_loop(0, PH, chunk, 0)
        return c

    lax.fori_loop(0, nph, phase, 0)
    plsc.subcore_barrier()
    pltpu.sync_copy(dsh.at[pl.ds(sid * RPS, RPS)],
                    out_hbm.at[pl.ds(cid * NP + sid * RPS, RPS)])


@functools.cache
def _deg_kernel():
    return pl.kernel(
        _deg_body,
        out_type=jax.ShapeDtypeStruct((2 * NP,), F32),
        mesh=_mesh(),
        scratch_types=[
            pltpu.VMEM((PH, CH), jnp.int32),
            pltpu.VMEM((CH,), F32),
            pltpu.VMEM((RPS,), F32),
            pltpu.VMEM_SHARED((NP,), F32),
        ],
    )


def _sc_degree(dstp):
    return _deg_kernel()(dstp)


def _scat_body(y_hbm, src_hbm, dst_hbm, out_hbm,
               sidx, didx, rows0, rows1, zsh, sem0, sem1):
    cid, sid, cbase, nph = _edge_partition()

    def zrow(i, c):
        rows0[i // 8, pl.ds((i % 8) * 16, 16)] = jnp.zeros((16,), F32)
        return c

    lax.fori_loop(0, CH * 8, zrow, 0)

    def zcopy(k, c):
        pltpu.sync_copy(rows0, zsh.at[pl.ds(sid * RPS + k * CH, CH)])
        return c

    lax.fori_loop(0, RPS // CH, zcopy, 0)
    plsc.subcore_barrier()

    npair = PH // 2

    def phase(p, c):
        pltpu.sync_copy(src_hbm.at[pl.ds(cbase + p * PH, PH)], sidx)
        pltpu.sync_copy(dst_hbm.at[pl.ds(cbase + p * PH, PH)], didx)
        pltpu.async_copy(y_hbm.at[sidx.at[0]], rows0, sem0)

        def pair(g2, c2):
            g0 = 2 * g2
            pltpu.async_copy(y_hbm.at[sidx.at[g0 + 1]], rows1, sem1)
            pltpu.make_async_copy(y_hbm.at[pl.ds(0, CH)], rows0, sem0).wait()
            pltpu.sync_copy(rows0, zsh.at[didx.at[g0]], add=True)

            @pl.when(g2 != npair - 1)
            def _():
                pltpu.async_copy(y_hbm.at[sidx.at[g0 + 2]], rows0, sem0)

            pltpu.make_async_copy(y_hbm.at[pl.ds(0, CH)], rows1, sem1).wait()
            pltpu.sync_copy(rows1, zsh.at[didx.at[g0 + 1]], add=True)
            return c2

        lax.fori_loop(0, npair, pair, 0)
        return c

    lax.fori_loop(0, nph, phase, 0)
    plsc.subcore_barrier()
    pltpu.sync_copy(zsh.at[pl.ds(sid * RPS, RPS)],
                    out_hbm.at[pl.ds(cid * NP + sid * RPS, RPS)])


@functools.cache
def _scat_kernel():
    return pl.kernel(
        _scat_body,
        out_type=jax.ShapeDtypeStruct((2 * NP, HID), F32),
        mesh=_mesh(),
        scratch_types=[
            pltpu.VMEM((PH, CH), jnp.int32),
            pltpu.VMEM((PH, CH), jnp.int32),
            pltpu.VMEM((CH, HID), F32),
            pltpu.VMEM((CH, HID), F32),
            pltpu.VMEM_SHARED((NP, HID), F32),
            pltpu.SemaphoreType.DMA,
            pltpu.SemaphoreType.DMA,
        ],
    )


def _sc_scatter(y, srcp, dstp):
    return _scat_kernel()(y, srcp, dstp)


def _prep_body(deg_ref, nt_ref, emb_ref, w_ref, y_ref, dinv_ref):
    deg = deg_ref[0, :] + deg_ref[1, :] + 1.0
    dinv = lax.rsqrt(deg)
    table = jnp.dot(emb_ref[...], w_ref[...])
    nt = nt_ref[...]
    oh = (nt[:, None] == lax.broadcasted_iota(jnp.int32, (BLK, 4), 1)).astype(F32)
    x = jnp.dot(oh, table, precision=HIGH)
    y_ref[...] = x * dinv[:, None]
    dinv_ref[...] = dinv


def _tc_prep(deg2, ntp, node_emb, w1):
    return pl.pallas_call(
        _prep_body,
        grid=(NP // BLK,),
        in_specs=[
            pl.BlockSpec((2, BLK), lambda i: (0, i)),
            pl.BlockSpec((BLK,), lambda i: (i,)),
            pl.BlockSpec((4, HID), lambda i: (0, 0)),
            pl.BlockSpec((HID, HID), lambda i: (0, 0)),
        ],
        out_specs=[
            pl.BlockSpec((BLK, HID), lambda i: (i, 0)),
            pl.BlockSpec((BLK,), lambda i: (i,)),
        ],
        out_shape=[
            jax.ShapeDtypeStruct((NP, HID), F32),
            jax.ShapeDtypeStruct((NP,), F32),
        ],
    )(deg2, ntp, node_emb, w1)


def _mid_body(z0_ref, z1_ref, y_ref, dinv_ref, b_ref, w_ref, yn_ref):
    dv = dinv_ref[...]
    x = jnp.maximum(
        dv[:, None] * (z0_ref[...] + z1_ref[...] + y_ref[...])
        + b_ref[...][None, :], 0.0)
    yn_ref[...] = jnp.dot(x, w_ref[...]) * dv[:, None]


def _tc_mid(z0, z1, y, dinv, b, w_next):
    return pl.pallas_call(
        _mid_body,
        grid=(NP // BLK,),
        in_specs=[
            pl.BlockSpec((BLK, HID), lambda i: (i, 0)),
            pl.BlockSpec((BLK, HID), lambda i: (i, 0)),
            pl.BlockSpec((BLK, HID), lambda i: (i, 0)),
            pl.BlockSpec((BLK,), lambda i: (i,)),
            pl.BlockSpec((HID,), lambda i: (0,)),
            pl.BlockSpec((HID, HID), lambda i: (0, 0)),
        ],
        out_specs=pl.BlockSpec((BLK, HID), lambda i: (i, 0)),
        out_shape=jax.ShapeDtypeStruct((NP, HID), F32),
    )(z0, z1, y, dinv, b, w_next)


def _head_body(z0s_ref, z1s_ref, ys_ref, dvs_ref,
               z0d_ref, z1d_ref, yd_ref, dvd_ref,
               b3_ref, wh1_ref, bh1_ref, wh2_ref, bh2_ref, out_ref):
    b3 = b3_ref[...][None, :]
    xs = jnp.maximum(
        dvs_ref[...][:, None] * (z0s_ref[...] + z1s_ref[...] + ys_ref[...]) + b3,
        0.0)
    xd = jnp.maximum(
        dvd_ref[...][:, None] * (z0d_ref[...] + z1d_ref[...] + yd_ref[...]) + b3,
        0.0)
    h = jnp.maximum(
        jnp.dot(xs, wh1_ref[0:HID, :])
        + jnp.dot(xd, wh1_ref[HID:2 * HID, :])
        + bh1_ref[...][None, :], 0.0)
    out_ref[...] = jnp.dot(h, wh2_ref[...]) + bh2_ref[...][None, :]


def _tc_head(bs, z0, z1, y, dinv, b3, wh1, bh1, wh2, bh2):
    return pl.pallas_call(
        _head_body,
        out_shape=jax.ShapeDtypeStruct((bs, 1), F32),
    )(z0[:bs], z1[:bs], y[:bs], dinv[:bs],
      z0[bs:2 * bs], z1[bs:2 * bs], y[bs:2 * bs], dinv[bs:2 * bs],
      b3, wh1, bh1, wh2, bh2)


def kernel(node_type, edge_type, edge_index, edge_label, node_emb, edge_emb,
           W1, b1, W2, b2, W3, b3, Wh1, bh1, Wh2, bh2):
    del edge_type, edge_emb
    src = edge_index[0].astype(jnp.int32)
    dst = edge_index[1].astype(jnp.int32)
    srcp = jnp.concatenate(
        [src, jnp.zeros((EPAD - E,), jnp.int32)]).reshape(TCH, CH)
    pad_dst = N + (jnp.arange(EPAD - E, dtype=jnp.int32) % NTRASH)
    dstp = jnp.concatenate([dst, pad_dst]).reshape(TCH, CH)
    ntp = jnp.pad(node_type.astype(jnp.int32), (0, NP - N))

    deg2 = _sc_degree(dstp).reshape(2, NP)
    y1, dinv = _tc_prep(deg2, ntp, node_emb, W1)

    z = _sc_scatter(y1, srcp, dstp)
    y2 = _tc_mid(z[:NP], z[NP:], y1, dinv, b1, W2)
    z = _sc_scatter(y2, srcp, dstp)
    y3 = _tc_mid(z[:NP], z[NP:], y2, dinv, b2, W3)
    z = _sc_scatter(y3, srcp, dstp)

    bs = edge_label.shape[0]
    pred = _tc_head(bs, z[:NP], z[NP:], y3, dinv, b3, Wh1, bh1, Wh2, bh2)
    return (pred, edge_label)


# speedup vs baseline: 3.1598x; 3.1598x over previous
import functools

import jax
import jax.numpy as jnp
from jax import lax
from jax.experimental import pallas as pl
from jax.experimental.pallas import tpu as pltpu
from jax.experimental.pallas import tpu_sc as plsc

N = 10000
NP = 10240
HID = 128
E = 320000
CH = 128
PT0 = 80
PT1 = 80
PH = 16
TCH = 16 * (PT0 + PT1)
EPAD = TCH * CH
NTRASH = 240
RPS = NP // 16
BLK = 1024
F32 = jnp.float32
HIGH = lax.Precision.HIGHEST

@functools.cache
def _mesh():
    return plsc.VectorSubcoreMesh(core_axis_name="c", subcore_axis_name="s",
                                  num_cores=2, num_subcores=16)


def _edge_partition():
    cid = lax.axis_index("c")
    sid = lax.axis_index("s")
    ptc = jnp.where(cid == 0, PT0, PT1)
    nph = ptc // PH
    cbase = cid * (16 * PT0) + sid * ptc
    return cid, sid, cbase, nph


def _deg_body(dst_hbm, out_hbm, didx, ones, zb, dsh):
    cid, sid, cbase, nph = _edge_partition()

    def init_ones(i, c):
        ones[pl.ds(i * 16, 16)] = jnp.ones((16,), F32)
        return c

    lax.fori_loop(0, CH // 16, init_ones, 0)

    def init_zb(i, c):
        zb[pl.ds(i * 16, 16)] = jnp.zeros((16,), F32)
        return c

    lax.fori_loop(0, RPS // 16, init_zb, 0)
    pltpu.sync_copy(zb, dsh.at[pl.ds(sid * RPS, RPS)])
    plsc.subcore_barrier()

    def phase(p, c):
        pltpu.sync_copy(dst_hbm.at[pl.ds(cbase + p * PH, PH)], didx)

        def chunk(g, c2):
            pltpu.sync_copy(ones, dsh.at[didx.at[g]], add=True)
            return c2

        lax.fori_loop(0, PH, chunk, 0)
        return c

    lax.fori_loop(0, nph, phase, 0)
    plsc.subcore_barrier()
    pltpu.sync_copy(dsh.at[pl.ds(sid * RPS, RPS)],
                    out_hbm.at[pl.ds(cid * NP + sid * RPS, RPS)])


@functools.cache
def _deg_kernel():
    return pl.kernel(
        _deg_body,
        out_type=jax.ShapeDtypeStruct((2 * NP,), F32),
        mesh=_mesh(),
        scratch_types=[
            pltpu.VMEM((PH, CH), jnp.int32),
            pltpu.VMEM((CH,), F32),
            pltpu.VMEM((RPS,), F32),
            pltpu.VMEM_SHARED((NP,), F32),
        ],
    )


def _sc_degree(dstp):
    return _deg_kernel()(dstp)


def _scat_body(y_hbm, src_hbm, dst_hbm, out_hbm,
               sidx, didx, rows0, rows1, zsh, sem0, sem1):
    cid, sid, cbase, nph = _edge_partition()

    def zrow(i, c):
        rows0[i // 8, pl.ds((i % 8) * 16, 16)] = jnp.zeros((16,), F32)
        return c

    lax.fori_loop(0, CH * 8, zrow, 0)

    def zcopy(k, c):
        pltpu.sync_copy(rows0, zsh.at[pl.ds(sid * RPS + k * CH, CH)])
        return c

    lax.fori_loop(0, RPS // CH, zcopy, 0)
    plsc.subcore_barrier()

    npair = PH // 2

    def phase(p, c):
        pltpu.sync_copy(src_hbm.at[pl.ds(cbase + p * PH, PH)], sidx)
        pltpu.sync_copy(dst_hbm.at[pl.ds(cbase + p * PH, PH)], didx)
        pltpu.async_copy(y_hbm.at[sidx.at[0]], rows0, sem0)

        def pair(g2, c2):
            g0 = 2 * g2
            pltpu.async_copy(y_hbm.at[sidx.at[g0 + 1]], rows1, sem1)
            pltpu.make_async_copy(y_hbm.at[pl.ds(0, CH)], rows0, sem0).wait()
            pltpu.sync_copy(rows0, zsh.at[didx.at[g0]], add=True)

            @pl.when(g2 != npair - 1)
            def _():
                pltpu.async_copy(y_hbm.at[sidx.at[g0 + 2]], rows0, sem0)

            pltpu.make_async_copy(y_hbm.at[pl.ds(0, CH)], rows1, sem1).wait()
            pltpu.sync_copy(rows1, zsh.at[didx.at[g0 + 1]], add=True)
            return c2

        lax.fori_loop(0, npair, pair, 0)
        return c

    lax.fori_loop(0, nph, phase, 0)
    plsc.subcore_barrier()
    pltpu.sync_copy(zsh.at[pl.ds(sid * RPS, RPS)],
                    out_hbm.at[pl.ds(cid * NP + sid * RPS, RPS)])


@functools.cache
def _scat_kernel():
    return pl.kernel(
        _scat_body,
        out_type=jax.ShapeDtypeStruct((2 * NP, HID), F32),
        mesh=_mesh(),
        scratch_types=[
            pltpu.VMEM((PH, CH), jnp.int32),
            pltpu.VMEM((PH, CH), jnp.int32),
            pltpu.VMEM((CH, HID), F32),
            pltpu.VMEM((CH, HID), F32),
            pltpu.VMEM_SHARED((NP, HID), F32),
            pltpu.SemaphoreType.DMA,
            pltpu.SemaphoreType.DMA,
        ],
    )


def _sc_scatter(y, srcp, dstp):
    return _scat_kernel()(y, srcp, dstp)


def _prep_body(deg_ref, nt_ref, emb_ref, w_ref, y_ref, dinv_ref):
    deg = deg_ref[0, :] + deg_ref[1, :] + 1.0
    dinv = lax.rsqrt(deg)
    table = jnp.dot(emb_ref[...], w_ref[...])
    nt = nt_ref[...]
    oh = (nt[:, None] == lax.broadcasted_iota(jnp.int32, (BLK, 4), 1)).astype(F32)
    x = jnp.dot(oh, table, precision=HIGH)
    y_ref[...] = x * dinv[:, None]
    dinv_ref[...] = dinv


def _tc_prep(deg2, ntp, node_emb, w1):
    return pl.pallas_call(
        _prep_body,
        grid=(NP // BLK,),
        in_specs=[
            pl.BlockSpec((2, BLK), lambda i: (0, i)),
            pl.BlockSpec((BLK,), lambda i: (i,)),
            pl.BlockSpec((4, HID), lambda i: (0, 0)),
            pl.BlockSpec((HID, HID), lambda i: (0, 0)),
        ],
        out_specs=[
            pl.BlockSpec((BLK, HID), lambda i: (i, 0)),
            pl.BlockSpec((BLK,), lambda i: (i,)),
        ],
        out_shape=[
            jax.ShapeDtypeStruct((NP, HID), F32),
            jax.ShapeDtypeStruct((NP,), F32),
        ],
    )(deg2, ntp, node_emb, w1)


def _mid_body(z0_ref, z1_ref, y_ref, dinv_ref, b_ref, w_ref, yn_ref):
    dv = dinv_ref[...]
    x = jnp.maximum(
        dv[:, None] * (z0_ref[...] + z1_ref[...] + y_ref[...])
        + b_ref[...][None, :], 0.0)
    yn_ref[...] = jnp.dot(x, w_ref[...]) * dv[:, None]


def _tc_mid(z0, z1, y, dinv, b, w_next):
    return pl.pallas_call(
        _mid_body,
        grid=(NP // BLK,),
        in_specs=[
            pl.BlockSpec((BLK, HID), lambda i: (i, 0)),
            pl.BlockSpec((BLK, HID), lambda i: (i, 0)),
            pl.BlockSpec((BLK, HID), lambda i: (i, 0)),
            pl.BlockSpec((BLK,), lambda i: (i,)),
            pl.BlockSpec((HID,), lambda i: (0,)),
            pl.BlockSpec((HID, HID), lambda i: (0, 0)),
        ],
        out_specs=pl.BlockSpec((BLK, HID), lambda i: (i, 0)),
        out_shape=jax.ShapeDtypeStruct((NP, HID), F32),
    )(z0, z1, y, dinv, b, w_next)


def _head_body(z0s_ref, z1s_ref, ys_ref, dvs_ref,
               z0d_ref, z1d_ref, yd_ref, dvd_ref,
               b3_ref, wh1_ref, bh1_ref, wh2_ref, bh2_ref, out_ref):
    b3 = b3_ref[...][None, :]
    xs = jnp.maximum(
        dvs_ref[...][:, None] * (z0s_ref[...] + z1s_ref[...] + ys_ref[...]) + b3,
        0.0)
    xd = jnp.maximum(
        dvd_ref[...][:, None] * (z0d_ref[...] + z1d_ref[...] + yd_ref[...]) + b3,
        0.0)
    h = jnp.maximum(
        jnp.dot(xs, wh1_ref[0:HID, :])
        + jnp.dot(xd, wh1_ref[HID:2 * HID, :])
        + bh1_ref[...][None, :], 0.0)
    out_ref[...] = jnp.dot(h, wh2_ref[...]) + bh2_ref[...][None, :]


def _tc_head(bs, z0, z1, y, dinv, b3, wh1, bh1, wh2, bh2):
    return pl.pallas_call(
        _head_body,
        out_shape=jax.ShapeDtypeStruct((bs, 1), F32),
    )(z0[:bs], z1[:bs], y[:bs], dinv[:bs],
      z0[bs:2 * bs], z1[bs:2 * bs], y[bs:2 * bs], dinv[bs:2 * bs],
      b3, wh1, bh1, wh2, bh2)


def kernel(node_type, edge_type, edge_index, edge_label, node_emb, edge_emb,
           W1, b1, W2, b2, W3, b3, Wh1, bh1, Wh2, bh2):
    del edge_type, edge_emb
    src = edge_index[0].astype(jnp.int32)
    dst = edge_index[1].astype(jnp.int32)
    pad_src = jnp.arange(EPAD - E, dtype=jnp.int32) % N
    srcp = jnp.concatenate([src, pad_src]).reshape(TCH, CH)
    pad_dst = N + (jnp.arange(EPAD - E, dtype=jnp.int32) % NTRASH)
    dstp = jnp.concatenate([dst, pad_dst]).reshape(TCH, CH)
    ntp = jnp.pad(node_type.astype(jnp.int32), (0, NP - N))

    deg2 = _sc_degree(dstp).reshape(2, NP)
    y1, dinv = _tc_prep(deg2, ntp, node_emb, W1)

    z = _sc_scatter(y1, srcp, dstp)
    y2 = _tc_mid(z[:NP], z[NP:], y1, dinv, b1, W2)
    z = _sc_scatter(y2, srcp, dstp)
    y3 = _tc_mid(z[:NP], z[NP:], y2, dinv, b2, W3)
    z = _sc_scatter(y3, srcp, dstp)

    bs = edge_label.shape[0]
    pred = _tc_head(bs, z[:NP], z[NP:], y3, dinv, b3, Wh1, bh1, Wh2, bh2)
    return (pred, edge_label)
